# baseline (speedup 1.0000x reference)
import functools

import jax
import jax.numpy as jnp
from jax.experimental import pallas as pl
from jax.experimental.pallas import tpu as pltpu

_T, _N, _C = 512, 32, 1280
_NS, _NA = 256, 5
_SL = 4
_NB = 4
_LW = 256
_NEGBIG = -1e38


_NG = 8
_TCH = 128


def _gather_body(idx_ref, scores_ref, out_ref):
    cols = jax.lax.broadcasted_iota(jnp.int32, (_C, 2 * _LW), 0)
    for i in range(_NG):
        idx = idx_ref[i, 0, :]
        onehot = jnp.where(cols == idx[None, :], 1.0, 0.0).astype(jnp.bfloat16)
        sb = scores_ref[:, i, :].astype(jnp.bfloat16)
        out_ref[:, i, :] = jnp.dot(sb, onehot, preferred_element_type=jnp.float32)


def _fused_scan_body(lens_ref, scores_ref, gath_ref, out_ref, acrf_ref, actc_ref):
    t = pl.program_id(0)
    lane_s = jax.lax.broadcasted_iota(jnp.int32, (_N, _NS), 1)
    lane_l = jax.lax.broadcasted_iota(jnp.int32, (_N, _LW), 1)

    @pl.when(t == 0)
    def _init():
        acrf_ref[...] = jnp.zeros((_N, _NS), jnp.float32)
        actc_ref[...] = jnp.where(lane_l == 0, 0.0, _NEGBIG)

    s = scores_ref[0]
    M = s.reshape(_N, _NS, _NA)

    alpha = acrf_ref[...]
    a4 = alpha.reshape(_N, _NB, _NS // _NB)
    rep = jnp.repeat(a4, _NB, axis=2)
    t0 = M[:, :, 0] + alpha
    terms = [M[:, :, k + 1] + rep[:, k, :] for k in range(_NB)]
    m = t0
    for x in terms:
        m = jnp.maximum(m, x)
    acc = jnp.exp(t0 - m)
    for x in terms:
        acc = acc + jnp.exp(x - m)
    acrf_ref[...] = m + jnp.log(acc)

    g = gath_ref[0]
    st = jnp.where(lane_l < _LW - 3, g[:, :_LW], 0.0)
    mv = jnp.where(
        jnp.logical_and(lane_l >= 1, lane_l < _LW - 3), g[:, _LW:], _NEGBIG
    )
    actc = actc_ref[...]
    shifted = jnp.where(lane_l == 0, _NEGBIG, pltpu.roll(actc, 1, axis=1))
    x1 = actc + st
    x2 = shifted + mv
    mm = jnp.maximum(x1, x2)
    actc_ref[...] = mm + jnp.log(jnp.exp(x1 - mm) + jnp.exp(x2 - mm))

    @pl.when(t == _T - 1)
    def _fin():
        af = acrf_ref[...]
        mc = jnp.max(af, axis=1, keepdims=True)
        logz_crf = mc + jnp.log(
            jnp.sum(jnp.exp(af - mc), axis=1, keepdims=True)
        )
        ac = actc_ref[...]
        lens = lens_ref[...]
        picked = jnp.max(
            jnp.where(lane_l == lens - 1, ac, _NEGBIG), axis=1, keepdims=True
        )
        tl = (lens + (_SL - 1)).astype(jnp.float32)
        loss = -(picked - logz_crf) / tl
        out_ref[...] = jnp.broadcast_to(jnp.sum(loss) / _N, (8, 128))


def kernel(scores, targets, target_lengths):
    T, N, C = scores.shape
    L = targets.shape[1]
    n = L - (_SL - 1)

    tg = jnp.clip(targets - 1, 0, None)
    stay_idx = sum(
        tg[:, i : n + i] * (_NB ** (_SL - i - 1)) for i in range(_SL)
    ) * _NA
    move_idx = stay_idx[:, 1:] + tg[:, : n - 1] + 1
    pad_st = jnp.full((N, _LW - n), _C, jnp.int32)
    pad_mv = jnp.full((N, _LW - n + 1), _C, jnp.int32)
    idx = jnp.concatenate(
        [stay_idx.astype(jnp.int32), pad_st, pad_mv[:, :1], move_idx.astype(jnp.int32), pad_mv[:, 1:]],
        axis=1,
    )
    idx = idx.reshape(N, 1, 2 * _LW)

    gathered = pl.pallas_call(
        _gather_body,
        grid=(N // _NG, T // _TCH),
        in_specs=[
            pl.BlockSpec((_NG, 1, 2 * _LW), lambda g, tc: (g, 0, 0)),
            pl.BlockSpec((_TCH, _NG, C), lambda g, tc: (tc, g, 0)),
        ],
        out_specs=pl.BlockSpec((_TCH, _NG, 2 * _LW), lambda g, tc: (tc, g, 0)),
        out_shape=jax.ShapeDtypeStruct((T, N, 2 * _LW), jnp.float32),
    )(idx, scores)

    lens = (target_lengths + 1 - _SL).astype(jnp.int32).reshape(N, 1)

    out = pl.pallas_call(
        _fused_scan_body,
        grid=(T,),
        in_specs=[
            pl.BlockSpec((N, 1), lambda t: (0, 0)),
            pl.BlockSpec((1, N, C), lambda t: (t, 0, 0)),
            pl.BlockSpec((1, N, 2 * _LW), lambda t: (t, 0, 0)),
        ],
        out_specs=pl.BlockSpec((8, 128), lambda t: (0, 0)),
        out_shape=jax.ShapeDtypeStruct((8, 128), jnp.float32),
        scratch_shapes=[
            pltpu.VMEM((_N, _NS), jnp.float32),
            pltpu.VMEM((_N, _LW), jnp.float32),
        ],
    )(lens, scores, gathered)

    return out[0, 0]


# speedup vs baseline: 43.4706x; 43.4706x over previous
import jax
import jax.numpy as jnp
from jax.experimental import pallas as pl
from jax.experimental.pallas import tpu as pltpu

_T, _N, _C = 512, 32, 1280
_NS, _NA = 256, 5
_SL = 4
_NB = 4
_LW = 256
_H = 128
_NEGBIG = -1e38


def _gather256(plane, idx):
    lo = jnp.take_along_axis(plane[:, :_H], idx & (_H - 1), axis=1)
    hi = jnp.take_along_axis(plane[:, _H:], idx & (_H - 1), axis=1)
    return jnp.where(idx < _H, lo, hi)


_TB = 16


def _scan_body(lens_ref, sidx_ref, dsel_ref, planes_ref, out_ref,
               acrf_ref, actc_ref):
    t = pl.program_id(0)
    lane_l = jax.lax.broadcasted_iota(jnp.int32, (_N, _LW), 1)

    @pl.when(t == 0)
    def _init():
        acrf_ref[...] = jnp.zeros((_N, _NS), jnp.float32)
        actc_ref[...] = jnp.where(lane_l == 0, 0.0, _NEGBIG)

    lane_q = jax.lax.broadcasted_iota(jnp.int32, (_N, _NS), 1) >> 2
    sidx = sidx_ref[...]
    dsel = dsel_ref[...]
    stmask = lane_l < _LW - 3

    alpha = acrf_ref[...]
    actc = actc_ref[...]
    for tb in range(_TB):
        p = [planes_ref[tb, j] for j in range(_NA)]

        t0 = p[0] + alpha
        m = t0
        terms = []
        for k in range(_NB):
            half = alpha[:, : _H] if k < 2 else alpha[:, _H:]
            src = jnp.take_along_axis(
                half, lane_q + (k % 2) * (_NS // _NB), axis=1
            )
            x = p[k + 1] + src
            terms.append(x)
            m = jnp.maximum(m, x)
        acc = jnp.exp(t0 - m)
        for x in terms:
            acc = acc + jnp.exp(x - m)
        alpha = m + jnp.log(acc)

        st = jnp.where(stmask, _gather256(p[0], sidx), 0.0)
        mv = _NEGBIG * jnp.ones((_N, _LW), jnp.float32)
        for d in range(1, _NA):
            mv = jnp.where(dsel == d, _gather256(p[d], sidx), mv)
        shifted = jnp.where(lane_l == 0, _NEGBIG, pltpu.roll(actc, 1, axis=1))
        x1 = actc + st
        x2 = shifted + mv
        mm = jnp.maximum(x1, x2)
        actc = mm + jnp.log(jnp.exp(x1 - mm) + jnp.exp(x2 - mm))
    acrf_ref[...] = alpha
    actc_ref[...] = actc

    @pl.when(t == _T // _TB - 1)
    def _fin():
        af = acrf_ref[...]
        mc = jnp.max(af, axis=1, keepdims=True)
        logz_crf = mc + jnp.log(
            jnp.sum(jnp.exp(af - mc), axis=1, keepdims=True)
        )
        ac = actc_ref[...]
        lens = lens_ref[...]
        picked = jnp.max(
            jnp.where(lane_l == lens - 1, ac, _NEGBIG), axis=1, keepdims=True
        )
        tl = (lens + (_SL - 1)).astype(jnp.float32)
        loss = -(picked - logz_crf) / tl
        out_ref[...] = jnp.broadcast_to(jnp.sum(loss) / _N, (8, 128))


def kernel(scores, targets, target_lengths):
    T, N, C = scores.shape
    L = targets.shape[1]
    n = L - (_SL - 1)

    planes = jnp.transpose(scores.reshape(T, N, _NS, _NA), (0, 3, 1, 2))

    tg = jnp.clip(targets - 1, 0, None)
    state = sum(
        tg[:, i : n + i] * (_NB ** (_SL - i - 1)) for i in range(_SL)
    ).astype(jnp.int32)
    sidx = jnp.concatenate(
        [state, jnp.zeros((N, _LW - n), jnp.int32)], axis=1
    )
    dsel = jnp.concatenate(
        [
            jnp.zeros((N, 1), jnp.int32),
            tg[:, : n - 1].astype(jnp.int32) + 1,
            jnp.zeros((N, _LW - n), jnp.int32),
        ],
        axis=1,
    )

    lens = (target_lengths + 1 - _SL).astype(jnp.int32).reshape(N, 1)

    out = pl.pallas_call(
        _scan_body,
        grid=(T // _TB,),
        in_specs=[
            pl.BlockSpec((N, 1), lambda t: (0, 0)),
            pl.BlockSpec((N, _LW), lambda t: (0, 0)),
            pl.BlockSpec((N, _LW), lambda t: (0, 0)),
            pl.BlockSpec((_TB, _NA, N, _NS), lambda t: (t, 0, 0, 0)),
        ],
        out_specs=pl.BlockSpec((8, 128), lambda t: (0, 0)),
        out_shape=jax.ShapeDtypeStruct((8, 128), jnp.float32),
        scratch_shapes=[
            pltpu.VMEM((_N, _NS), jnp.float32),
            pltpu.VMEM((_N, _LW), jnp.float32),
        ],
    )(lens, sidx, dsel, planes)

    return out[0, 0]
